# Initial kernel scaffold; baseline (speedup 1.0000x reference)
import jax
import jax.numpy as jnp
from jax.experimental import pallas as pl


def kernel(x, a, b, alpha, id_gain, bias):
    raise NotImplementedError("write your pallas kernel here")


# speedup vs baseline: 3415.0867x; 3415.0867x over previous
import functools

import numpy as np
import jax
import jax.numpy as jnp
from jax import lax
from jax.experimental import pallas as pl
from jax.experimental.pallas import tpu as pltpu
from jax.experimental.pallas import tpu_sc as plsc

_C = 192
_K = 16
_P = 3
_CLAMP = 1.5
_NSPAN = _K - _P
_REC = 64
_TBL_WORDS = _C * _REC
_INVH = 6.5


def _knots_f64():
    n_int = _K - _P - 1
    interior = np.linspace(-1.0, 1.0, n_int + 2)[1:-1]
    return np.concatenate(
        [np.full(_P + 1, -1.0), interior, np.full(_P + 1, 1.0)])


def _local_basis_f64(x, i, kn):
    js = np.arange(1, _P + 1)
    left = x - kn[i + 1 - js]
    right = kn[i + js] - x
    N = np.zeros(_P + 1)
    N[0] = 1.0
    for j in range(1, _P + 1):
        saved = 0.0
        for r in range(j):
            denom = right[r] + left[j - r - 1]
            temp = N[r] / denom
            N[r] = saved + right[r] * temp
            saved = left[j - r - 1] * temp
        N[j] = saved
    return N


def _basis_matrices():
    kn = _knots_f64()
    M = np.zeros((_K, 4 * _NSPAN))
    for s in range(_NSPAN):
        i = s + _P
        t0, t1 = kn[i], kn[i + 1]
        xs = t0 + (t1 - t0) * np.array([0.1, 0.35, 0.65, 0.9])
        V = np.vander(xs, 4, increasing=True)
        Nm = np.stack([_local_basis_f64(x, i, kn) for x in xs])
        Bj = np.linalg.solve(V, Nm).T
        for j in range(4):
            M[s + j, s * 4:s * 4 + 4] = Bj[j]
    mask = np.zeros((1, 4 * _NSPAN))
    mask[0, ::4] = 1.0
    return M.astype(np.float32), mask.astype(np.float32)


_M_NP, _MSK_NP = _basis_matrices()


def _prep_body(alpha_ref, scal_ref, mat_ref, msk_ref, out_ref):
    alpha = alpha_ref[:]
    coef = jnp.dot(alpha, mat_ref[:], preferred_element_type=jnp.float32,
                   precision=lax.Precision.HIGHEST)
    coef = coef + scal_ref[:, 3:4] * msk_ref[:]
    pad = jnp.zeros((_C, _REC - 4 * _NSPAN - 3), jnp.float32)
    out_ref[:] = jnp.concatenate(
        [coef, scal_ref[:, 0:1], scal_ref[:, 1:2], scal_ref[:, 2:3], pad],
        axis=1)


def _prep_table(alpha, scal):
    return pl.pallas_call(
        _prep_body,
        out_shape=jax.ShapeDtypeStruct((_C, _REC), jnp.float32),
    )(alpha, scal, jnp.asarray(_M_NP), jnp.asarray(_MSK_NP))


def _make_sc_kernel(total):
    slice_elems = 224 * 224
    cps = 4
    chunk = slice_elems // cps
    nw = 32
    chunks = total // chunk
    assert chunks % nw == 0
    cpw = chunks // nw
    vregs = chunk // 16

    mesh = plsc.VectorSubcoreMesh(core_axis_name="c", subcore_axis_name="s")

    @functools.partial(
        pl.kernel,
        out_type=jax.ShapeDtypeStruct((total,), jnp.float32),
        mesh=mesh,
        scratch_types=[
            pltpu.VMEM((_TBL_WORDS,), jnp.float32),
            pltpu.VMEM((chunk,), jnp.float32),
            pltpu.VMEM((chunk,), jnp.float32),
        ],
        compiler_params=pltpu.CompilerParams(needs_layout_passes=False),
    )
    def sc_kernel(x_hbm, tbl_hbm, out_hbm, tbl_v, xv, yv):
        wid = lax.axis_index("s") * 2 + lax.axis_index("c")
        pltpu.sync_copy(tbl_hbm, tbl_v)
        g0 = wid * cpw

        @pl.loop(0, cpw)
        def _chunk_loop(k):
            g = g0 + k
            off = g * chunk
            pltpu.sync_copy(x_hbm.at[pl.ds(off, chunk)], xv)
            c = (g // cps) % _C
            bvec = jnp.full((16,), c * _REC, jnp.int32)
            av = plsc.load_gather(tbl_v, [bvec + 52])
            bv = plsc.load_gather(tbl_v, [bvec + 53])
            gv = plsc.load_gather(tbl_v, [bvec + 54])

            @pl.loop(0, vregs)
            def _vloop(i):
                xx = xv[pl.ds(i * 16, 16)]
                xa = jnp.minimum(jnp.maximum(xx * av + bv, -_CLAMP), _CLAMP)
                t = jnp.minimum(jnp.maximum(xa * _INVH + _INVH, 0.0), 12.0)
                s = t.astype(jnp.int32)
                i0 = s * 4 + bvec
                c0 = plsc.load_gather(tbl_v, [i0])
                c1 = plsc.load_gather(tbl_v, [i0 + 1])
                c2 = plsc.load_gather(tbl_v, [i0 + 2])
                c3 = plsc.load_gather(tbl_v, [i0 + 3])
                r = ((c3 * xa + c2) * xa + c1) * xa + c0
                yv[pl.ds(i * 16, 16)] = xx * gv + r

            pltpu.sync_copy(yv, out_hbm.at[pl.ds(off, chunk)])

    return sc_kernel


def kernel(x, a, b, alpha, id_gain, bias):
    scal = jnp.stack([a, b, id_gain, bias], axis=1)
    tbl = _prep_table(alpha, scal)
    total = x.size
    sc = _make_sc_kernel(total)
    y = sc(x.reshape(total), tbl.reshape(_TBL_WORDS))
    return y.reshape(x.shape)
